# baseline (speedup 1.0000x reference)
import jax
import jax.numpy as jnp
import numpy as np
from jax.experimental import pallas as pl
from jax.experimental.pallas import tpu as pltpu

D_MODEL = 1024
MAX_LEN = 5000
SLICE = D_MODEL // 4
TBL = 128
OFFS = (0, 24, 56, 63)
ROWS_PER_BLOCK = 512


def _positional_encoding(S, D):
    position = np.arange(S, dtype=np.float32)[:, None]
    div_term = np.exp(np.arange(0, D, 2, dtype=np.float32) * (-np.log(10000.0) / D))
    arg = (position * div_term).astype(np.float32)
    pe = np.stack([np.sin(arg), np.cos(arg)], axis=-1).reshape(S, D)
    return jnp.asarray(pe.astype(np.float32))


def _fused_kernel(h_ref, d_ref, wd_ref, mo_ref, x_ref, pe_ref,
                  he_ref, de_ref, we_ref, me_ref, w_ref, b_ref, g_ref, bb_ref,
                  o_ref, tbl_ref, *, rows, pe_blocks):
    i = pl.program_id(0)

    @pl.when(i == 0)
    def _pack_tables():
        tbl_ref[...] = jnp.zeros((TBL, D_MODEL), dtype=jnp.bfloat16)
        for k, (t, off) in enumerate(
                zip((he_ref, de_ref, we_ref, me_ref), OFFS)):
            n = t.shape[0]
            tbl_ref[off:off + n, k * SLICE:(k + 1) * SLICE] = (
                t[...].astype(jnp.bfloat16))

    b = i // pe_blocks
    s0 = (i % pe_blocks) * rows

    iota = jax.lax.broadcasted_iota(jnp.int32, (TBL, rows), 0)
    oh = jnp.zeros((TBL, rows), dtype=jnp.bfloat16)
    for off, r in zip(OFFS, (h_ref, d_ref, wd_ref, mo_ref)):
        idx = r[pl.ds(b, 1), pl.ds(s0, rows)]
        oh = oh + (idx + off == iota).astype(jnp.bfloat16)

    feats = jax.lax.dot_general(
        oh, tbl_ref[...], (((0,), (0,)), ((), ())),
        preferred_element_type=jnp.float32)

    emb = x_ref[...] + pe_ref[pl.ds(s0, rows), :] + feats

    z = jax.lax.dot_general(
        emb, w_ref[...], (((1,), (1,)), ((), ())),
        preferred_element_type=jnp.float32) + b_ref[...]

    mean = jnp.mean(z, axis=1, keepdims=True)
    c = z - mean
    var = jnp.mean(c * c, axis=1, keepdims=True)
    o_ref[...] = c * jax.lax.rsqrt(var + 1e-5) * g_ref[...] + bb_ref[...]


def kernel(input_embeddings, hour, day, weekday, month, hour_emb, day_emb,
           weekday_emb, month_emb, W_proj, b_proj, ln_gamma, ln_beta):
    B, S, D = input_embeddings.shape
    N = B * S
    rows = ROWS_PER_BLOCK
    assert N % rows == 0 and S % rows == 0
    grid = N // rows
    pe_blocks = S // rows

    x = input_embeddings.reshape(N, D)
    pe = _positional_encoding(S, D)

    full = lambda shape: pl.BlockSpec(shape, lambda i: tuple(0 for _ in shape))

    out = pl.pallas_call(
        lambda *refs: _fused_kernel(*refs, rows=rows, pe_blocks=pe_blocks),
        grid=(grid,),
        in_specs=[
            full((B, S)),
            full((B, S)),
            full((B, S)),
            full((B, S)),
            pl.BlockSpec((rows, D), lambda i: (i, 0)),
            full((S, D)),
            full(hour_emb.shape),
            full(day_emb.shape),
            full(weekday_emb.shape),
            full(month_emb.shape),
            full((D, D)),
            full((1, D)),
            full((1, D)),
            full((1, D)),
        ],
        out_specs=pl.BlockSpec((rows, D), lambda i: (i, 0)),
        out_shape=jax.ShapeDtypeStruct((N, D), jnp.float32),
        scratch_shapes=[pltpu.VMEM((TBL, D), jnp.bfloat16)],
        compiler_params=pltpu.CompilerParams(
            dimension_semantics=("arbitrary",)),
    )(hour.astype(jnp.int32), day.astype(jnp.int32),
      weekday.astype(jnp.int32), month.astype(jnp.int32),
      x, pe, hour_emb, day_emb, weekday_emb, month_emb, W_proj,
      b_proj.reshape(1, D), ln_gamma.reshape(1, D), ln_beta.reshape(1, D))

    return out.reshape(B, S, D)


# speedup vs baseline: 1.0158x; 1.0158x over previous
import jax
import jax.numpy as jnp
import numpy as np
from jax.experimental import pallas as pl
from jax.experimental.pallas import tpu as pltpu

D_MODEL = 1024
MAX_LEN = 5000
SLICE = D_MODEL // 4
TBL = 128
OFFS = (0, 24, 56, 63)
ROWS_PER_BLOCK = 2048


def _positional_encoding(S, D):
    position = np.arange(S, dtype=np.float32)[:, None]
    div_term = np.exp(np.arange(0, D, 2, dtype=np.float32) * (-np.log(10000.0) / D))
    arg = (position * div_term).astype(np.float32)
    pe = np.stack([np.sin(arg), np.cos(arg)], axis=-1).reshape(S, D)
    return jnp.asarray(pe.astype(np.float32))


def _fused_kernel(h_ref, d_ref, wd_ref, mo_ref, x_ref, pe_ref,
                  he_ref, de_ref, we_ref, me_ref, w_ref, b_ref, g_ref, bb_ref,
                  o_ref, tbl_ref, *, rows, pe_blocks):
    i = pl.program_id(0)

    @pl.when(i == 0)
    def _pack_tables():
        tbl_ref[...] = jnp.zeros((TBL, D_MODEL), dtype=jnp.bfloat16)
        for k, (t, off) in enumerate(
                zip((he_ref, de_ref, we_ref, me_ref), OFFS)):
            n = t.shape[0]
            tbl_ref[off:off + n, k * SLICE:(k + 1) * SLICE] = (
                t[...].astype(jnp.bfloat16))

    b = i // pe_blocks
    s0 = (i % pe_blocks) * rows

    iota = jax.lax.broadcasted_iota(jnp.int32, (TBL, rows), 0)
    oh = jnp.zeros((TBL, rows), dtype=jnp.bfloat16)
    for off, r in zip(OFFS, (h_ref, d_ref, wd_ref, mo_ref)):
        idx = r[pl.ds(b, 1), pl.ds(s0, rows)]
        oh = oh + (idx + off == iota).astype(jnp.bfloat16)

    feats = jax.lax.dot_general(
        oh, tbl_ref[...], (((0,), (0,)), ((), ())),
        preferred_element_type=jnp.float32)

    emb = x_ref[...] + pe_ref[pl.ds(s0, rows), :] + feats

    z = jax.lax.dot_general(
        emb, w_ref[...], (((1,), (1,)), ((), ())),
        preferred_element_type=jnp.float32) + b_ref[...]

    mean = jnp.mean(z, axis=1, keepdims=True)
    c = z - mean
    var = jnp.mean(c * c, axis=1, keepdims=True)
    o_ref[...] = c * jax.lax.rsqrt(var + 1e-5) * g_ref[...] + bb_ref[...]


def kernel(input_embeddings, hour, day, weekday, month, hour_emb, day_emb,
           weekday_emb, month_emb, W_proj, b_proj, ln_gamma, ln_beta):
    B, S, D = input_embeddings.shape
    N = B * S
    rows = ROWS_PER_BLOCK
    assert N % rows == 0 and S % rows == 0
    grid = N // rows
    pe_blocks = S // rows

    x = input_embeddings.reshape(N, D)
    pe = _positional_encoding(S, D)

    full = lambda shape: pl.BlockSpec(shape, lambda i: tuple(0 for _ in shape))

    out = pl.pallas_call(
        lambda *refs: _fused_kernel(*refs, rows=rows, pe_blocks=pe_blocks),
        grid=(grid,),
        in_specs=[
            full((B, S)),
            full((B, S)),
            full((B, S)),
            full((B, S)),
            pl.BlockSpec((rows, D), lambda i: (i, 0)),
            full((S, D)),
            full(hour_emb.shape),
            full(day_emb.shape),
            full(weekday_emb.shape),
            full(month_emb.shape),
            full((D, D)),
            full((1, D)),
            full((1, D)),
            full((1, D)),
        ],
        out_specs=pl.BlockSpec((rows, D), lambda i: (i, 0)),
        out_shape=jax.ShapeDtypeStruct((N, D), jnp.float32),
        scratch_shapes=[pltpu.VMEM((TBL, D), jnp.bfloat16)],
        compiler_params=pltpu.CompilerParams(
            dimension_semantics=("arbitrary",)),
    )(hour.astype(jnp.int32), day.astype(jnp.int32),
      weekday.astype(jnp.int32), month.astype(jnp.int32),
      x, pe, hour_emb, day_emb, weekday_emb, month_emb, W_proj,
      b_proj.reshape(1, D), ln_gamma.reshape(1, D), ln_beta.reshape(1, D))

    return out.reshape(B, S, D)


# speedup vs baseline: 1.1047x; 1.0875x over previous
import jax
import jax.numpy as jnp
import numpy as np
from jax.experimental import pallas as pl
from jax.experimental.pallas import tpu as pltpu

D_MODEL = 1024
MAX_LEN = 5000
SLICE = D_MODEL // 4
TBL = 128
OFFS = (0, 24, 56, 63)
ROWS_PER_BLOCK = 1024


def _positional_encoding(S, D):
    position = np.arange(S, dtype=np.float32)[:, None]
    div_term = np.exp(np.arange(0, D, 2, dtype=np.float32) * (-np.log(10000.0) / D))
    arg = (position * div_term).astype(np.float32)
    pe = np.stack([np.sin(arg), np.cos(arg)], axis=-1).reshape(S, D)
    return jnp.asarray(pe.astype(np.float32).astype(jnp.bfloat16))


def _fused_kernel(h_ref, d_ref, wd_ref, mo_ref, x_ref, pe_ref,
                  he_ref, de_ref, we_ref, me_ref, w_ref, b_ref, g_ref, bb_ref,
                  o_ref, tbl_ref, *, rows, pe_blocks):
    i = pl.program_id(0)

    @pl.when(i == 0)
    def _pack_tables():
        tbl_ref[...] = jnp.zeros((TBL, D_MODEL), dtype=jnp.bfloat16)
        for k, (t, off) in enumerate(
                zip((he_ref, de_ref, we_ref, me_ref), OFFS)):
            n = t.shape[0]
            tbl_ref[off:off + n, k * SLICE:(k + 1) * SLICE] = (
                t[...].astype(jnp.bfloat16))

    b = i // pe_blocks
    s0 = (i % pe_blocks) * rows

    iota = jax.lax.broadcasted_iota(jnp.int32, (TBL, rows), 0)
    oh = jnp.zeros((TBL, rows), dtype=jnp.bfloat16)
    for off, r in zip(OFFS, (h_ref, d_ref, wd_ref, mo_ref)):
        idx = r[pl.ds(b, 1), pl.ds(s0, rows)]
        oh = oh + (idx + off == iota).astype(jnp.bfloat16)

    feats = jax.lax.dot_general(
        oh, tbl_ref[...], (((0,), (0,)), ((), ())),
        preferred_element_type=jnp.float32)

    emb = x_ref[...] + pe_ref[pl.ds(s0, rows), :].astype(jnp.float32) + feats

    z = jax.lax.dot_general(
        emb, w_ref[...], (((1,), (1,)), ((), ())),
        preferred_element_type=jnp.float32) + b_ref[...]

    mean = jnp.mean(z, axis=1, keepdims=True)
    c = z - mean
    var = jnp.mean(c * c, axis=1, keepdims=True)
    o_ref[...] = c * jax.lax.rsqrt(var + 1e-5) * g_ref[...] + bb_ref[...]


def kernel(input_embeddings, hour, day, weekday, month, hour_emb, day_emb,
           weekday_emb, month_emb, W_proj, b_proj, ln_gamma, ln_beta):
    B, S, D = input_embeddings.shape
    N = B * S
    rows = ROWS_PER_BLOCK
    assert N % rows == 0 and S % rows == 0
    grid = N // rows
    pe_blocks = S // rows

    x = input_embeddings.reshape(N, D)
    pe = _positional_encoding(S, D)

    full = lambda shape: pl.BlockSpec(shape, lambda i: tuple(0 for _ in shape))

    out = pl.pallas_call(
        lambda *refs: _fused_kernel(*refs, rows=rows, pe_blocks=pe_blocks),
        grid=(grid,),
        in_specs=[
            full((B, S)),
            full((B, S)),
            full((B, S)),
            full((B, S)),
            pl.BlockSpec((rows, D), lambda i: (i, 0)),
            full((S, D)),
            full(hour_emb.shape),
            full(day_emb.shape),
            full(weekday_emb.shape),
            full(month_emb.shape),
            full((D, D)),
            full((1, D)),
            full((1, D)),
            full((1, D)),
        ],
        out_specs=pl.BlockSpec((rows, D), lambda i: (i, 0)),
        out_shape=jax.ShapeDtypeStruct((N, D), jnp.float32),
        scratch_shapes=[pltpu.VMEM((TBL, D), jnp.bfloat16)],
        compiler_params=pltpu.CompilerParams(
            dimension_semantics=("arbitrary",)),
    )(hour.astype(jnp.int32), day.astype(jnp.int32),
      weekday.astype(jnp.int32), month.astype(jnp.int32),
      x, pe, hour_emb, day_emb, weekday_emb, month_emb, W_proj,
      b_proj.reshape(1, D), ln_gamma.reshape(1, D), ln_beta.reshape(1, D))

    return out.reshape(B, S, D)


# speedup vs baseline: 1.1325x; 1.0252x over previous
import jax
import jax.numpy as jnp
import numpy as np
from jax.experimental import pallas as pl
from jax.experimental.pallas import tpu as pltpu

D_MODEL = 1024
MAX_LEN = 5000
SLICE = D_MODEL // 4
TBL = 128
OFFS = (0, 24, 56, 63)
ROWS_PER_BLOCK = 1024


def _positional_encoding(S, D):
    position = np.arange(S, dtype=np.float32)[:, None]
    div_term = np.exp(np.arange(0, D, 2, dtype=np.float32) * (-np.log(10000.0) / D))
    arg = (position * div_term).astype(np.float32)
    pe = np.stack([np.sin(arg), np.cos(arg)], axis=-1).reshape(S, D)
    return jnp.asarray(pe.astype(np.float32).astype(jnp.bfloat16))


def _fused_kernel(h_ref, d_ref, wd_ref, mo_ref, x_ref, pe_ref,
                  he_ref, de_ref, we_ref, me_ref, w_ref, b_ref, g_ref, bb_ref,
                  o_ref, tbl_ref, *, rows, pe_blocks):
    i = pl.program_id(0)

    @pl.when(i == 0)
    def _pack_tables():
        tbl_ref[...] = jnp.zeros((TBL, D_MODEL), dtype=jnp.bfloat16)
        for k, (t, off) in enumerate(
                zip((he_ref, de_ref, we_ref, me_ref), OFFS)):
            n = t.shape[0]
            tbl_ref[off:off + n, k * SLICE:(k + 1) * SLICE] = (
                t[...].astype(jnp.bfloat16))

    b = i // pe_blocks
    s0 = (i % pe_blocks) * rows

    iota = jax.lax.broadcasted_iota(jnp.int32, (TBL, rows), 0)
    oh = jnp.zeros((TBL, rows), dtype=jnp.bfloat16)
    for off, r in zip(OFFS, (h_ref, d_ref, wd_ref, mo_ref)):
        idx = r[pl.ds(b, 1), pl.ds(s0, rows)]
        oh = oh + (idx + off == iota).astype(jnp.bfloat16)

    feats = jax.lax.dot_general(
        oh, tbl_ref[...], (((0,), (0,)), ((), ())),
        preferred_element_type=jnp.float32)

    emb = x_ref[...] + pe_ref[pl.ds(s0, rows), :].astype(jnp.float32) + feats

    z = jax.lax.dot_general(
        emb, w_ref[...], (((1,), (1,)), ((), ())),
        preferred_element_type=jnp.float32) + b_ref[...]

    mean = jnp.mean(z, axis=1, keepdims=True)
    msq = jnp.mean(z * z, axis=1, keepdims=True)
    var = msq - mean * mean
    o_ref[...] = (z - mean) * jax.lax.rsqrt(var + 1e-5) * g_ref[...] + bb_ref[...]


def kernel(input_embeddings, hour, day, weekday, month, hour_emb, day_emb,
           weekday_emb, month_emb, W_proj, b_proj, ln_gamma, ln_beta):
    B, S, D = input_embeddings.shape
    N = B * S
    rows = ROWS_PER_BLOCK
    assert N % rows == 0 and S % rows == 0
    grid = N // rows
    pe_blocks = S // rows

    x = input_embeddings.reshape(N, D)
    pe = _positional_encoding(S, D)

    full = lambda shape: pl.BlockSpec(shape, lambda i: tuple(0 for _ in shape))

    out = pl.pallas_call(
        lambda *refs: _fused_kernel(*refs, rows=rows, pe_blocks=pe_blocks),
        grid=(grid,),
        in_specs=[
            full((B, S)),
            full((B, S)),
            full((B, S)),
            full((B, S)),
            pl.BlockSpec((rows, D), lambda i: (i, 0)),
            full((S, D)),
            full(hour_emb.shape),
            full(day_emb.shape),
            full(weekday_emb.shape),
            full(month_emb.shape),
            full((D, D)),
            full((1, D)),
            full((1, D)),
            full((1, D)),
        ],
        out_specs=pl.BlockSpec((rows, D), lambda i: (i, 0)),
        out_shape=jax.ShapeDtypeStruct((N, D), jnp.float32),
        scratch_shapes=[pltpu.VMEM((TBL, D), jnp.bfloat16)],
        compiler_params=pltpu.CompilerParams(
            dimension_semantics=("arbitrary",)),
    )(hour.astype(jnp.int32), day.astype(jnp.int32),
      weekday.astype(jnp.int32), month.astype(jnp.int32),
      x, pe, hour_emb, day_emb, weekday_emb, month_emb, W_proj,
      b_proj.reshape(1, D), ln_gamma.reshape(1, D), ln_beta.reshape(1, D))

    return out.reshape(B, S, D)
